# Initial kernel scaffold; baseline (speedup 1.0000x reference)
import jax
import jax.numpy as jnp
from jax.experimental import pallas as pl


def kernel(X, aspect_ids, W_embs, b_embs):
    raise NotImplementedError("write your pallas kernel here")


# speedup vs baseline: 3.1622x; 3.1622x over previous
import functools

import jax
import jax.numpy as jnp
from jax import lax
from jax.experimental import pallas as pl
from jax.experimental.pallas import tpu as pltpu
from jax.experimental.pallas import tpu_sc as plsc

INPUT_DIM = 128
D2 = 2 * INPUT_DIM
BATCH = 16384
NC, NS, L = 2, 16, 16
NW = NC * NS
BPW = BATCH // NW
C = 128
NCHUNK = BPW // C


def _body(x_hbm, ids_hbm, w_hbm, b_hbm, out_hbm,
          idx_v, w_v, x_v, bias_v, brow_v, out_v, sem):
    wid = lax.axis_index("s") * NC + lax.axis_index("c")
    base = wid * BPW

    for ci in range(NCHUNK):
        pltpu.sync_copy(ids_hbm.at[pl.ds(base + ci * C, C)], idx_v.at[ci])

    iota = lax.iota(jnp.int32, L)
    cols = [[iota * 2 + (2 * L * t + k) for k in range(2)] for t in range(INPUT_DIM // L)]
    lane_masks = [iota == j for j in range(L)]
    half_iota = iota // 2
    par_iota = iota % 2

    for ci in range(NCHUNK):
        for i in range(C // L):
            brow_v[pl.ds(i * L, L)] = lax.shift_right_logical(
                idx_v[ci, pl.ds(i * L, L)], 6)
        pltpu.async_copy(w_hbm.at[idx_v.at[ci]], w_v, sem).wait()
        pltpu.async_copy(b_hbm.at[brow_v], bias_v, sem).wait()
        pltpu.sync_copy(x_hbm.at[pl.ds(base + ci * C, C)], x_v)

        def group_body(g, carry, ci=ci):
            rows8 = g * 8 + half_iota
            ids8 = plsc.load_gather(idx_v, [jnp.full((L,), ci, jnp.int32), rows8])
            bcols = 2 * (ids8 & 63) + par_iota
            outvec = plsc.load_gather(bias_v, [rows8, bcols])
            for rr in range(8):
                r = g * 8 + rr
                rsplat = jnp.full((L,), r, jnp.int32)
                acc0 = jnp.zeros((L,), jnp.float32)
                acc1 = jnp.zeros((L,), jnp.float32)
                for t in range(INPUT_DIM // L):
                    xv = x_v[r, pl.ds(t * L, L)]
                    w0 = plsc.load_gather(w_v, [rsplat, cols[t][0]])
                    w1 = plsc.load_gather(w_v, [rsplat, cols[t][1]])
                    acc0 = acc0 + xv * w0
                    acc1 = acc1 + xv * w1
                outvec = jnp.where(lane_masks[2 * rr], outvec + jnp.sum(acc0), outvec)
                outvec = jnp.where(lane_masks[2 * rr + 1], outvec + jnp.sum(acc1), outvec)
            out_v[pl.ds(ci * C * 2 + g * L, L)] = outvec
            return carry

        lax.fori_loop(0, C // 8, group_body, 0)

    pltpu.sync_copy(out_v, out_hbm.at[pl.ds(base * 2, BPW * 2)])


@jax.jit
def _run(X, ids, W_embs, b_embs):
    mesh = plsc.VectorSubcoreMesh(core_axis_name="c", subcore_axis_name="s",
                                  num_cores=NC, num_subcores=NS)
    f = functools.partial(
        pl.kernel,
        out_type=jax.ShapeDtypeStruct((BATCH * 2,), jnp.float32),
        mesh=mesh,
        compiler_params=pltpu.CompilerParams(needs_layout_passes=False),
        scratch_types=[
            pltpu.VMEM((NCHUNK, C), jnp.int32),
            pltpu.VMEM((C, D2), jnp.float32),
            pltpu.VMEM((C, INPUT_DIM), jnp.float32),
            pltpu.VMEM((C, 128), jnp.float32),
            pltpu.VMEM((C,), jnp.int32),
            pltpu.VMEM((BPW * 2,), jnp.float32),
            pltpu.SemaphoreType.DMA,
        ],
    )(_body)
    return f(X, ids, W_embs, b_embs)


def kernel(X, aspect_ids, W_embs, b_embs):
    bflat = b_embs.reshape(-1)
    pad = (-bflat.shape[0]) % 128
    b2 = jnp.pad(bflat, (0, pad)).reshape(-1, 128)
    return _run(X, aspect_ids.astype(jnp.int32), W_embs, b2).reshape(BATCH, 2)


# speedup vs baseline: 3.4667x; 1.0963x over previous
import functools

import jax
import jax.numpy as jnp
from jax import lax
from jax.experimental import pallas as pl
from jax.experimental.pallas import tpu as pltpu
from jax.experimental.pallas import tpu_sc as plsc

INPUT_DIM = 128
D2 = 2 * INPUT_DIM
BATCH = 16384
NC, NS, L = 2, 16, 16
NW = NC * NS
BPW = BATCH // NW
C = 64
NCHUNK = BPW // C


def _body(x_hbm, ids_hbm, w_hbm, b_hbm, out_hbm,
          idx_v, w_v, x_v, bias_v, brow_v, out_v, sems):
    wid = lax.axis_index("s") * NC + lax.axis_index("c")
    base = wid * BPW

    for ci in range(NCHUNK):
        pltpu.sync_copy(ids_hbm.at[pl.ds(base + ci * C, C)], idx_v.at[ci])
    for ci in range(NCHUNK):
        for i in range(C // L):
            brow_v[ci, pl.ds(i * L, L)] = lax.shift_right_logical(
                idx_v[ci, pl.ds(i * L, L)], 6)

    iota = lax.iota(jnp.int32, L)
    cols = [[iota * 2 + (2 * L * t + k) for k in range(2)] for t in range(INPUT_DIM // L)]
    lane_masks = [iota == j for j in range(L)]
    half_iota = iota // 2
    par_iota = iota % 2

    def fire(ci, slot):
        w_c = pltpu.async_copy(w_hbm.at[idx_v.at[ci]], w_v.at[slot], sems[0].at[slot])
        b_c = pltpu.async_copy(b_hbm.at[brow_v.at[ci]], bias_v.at[slot], sems[1].at[slot])
        x_c = pltpu.async_copy(x_hbm.at[pl.ds(base + ci * C, C)], x_v.at[slot],
                               sems[2].at[slot])
        return (w_c, b_c, x_c)

    pending = {0: fire(0, 0)}
    for ci in range(NCHUNK):
        slot = ci % 2
        if ci + 1 < NCHUNK:
            pending[ci + 1] = fire(ci + 1, (ci + 1) % 2)
        for h in pending.pop(ci):
            h.wait()
        ci_splat = jnp.full((L,), ci, jnp.int32)
        slot_splat = jnp.full((L,), slot, jnp.int32)

        def group_body(g, carry, ci=ci, slot=slot, ci_splat=ci_splat,
                       slot_splat=slot_splat):
            rows8 = g * 8 + half_iota
            ids8 = plsc.load_gather(idx_v, [ci_splat, rows8])
            bcols = 2 * (ids8 & 63) + par_iota
            outvec = plsc.load_gather(bias_v, [slot_splat, rows8, bcols])
            for rr in range(8):
                r = g * 8 + rr
                rsplat = jnp.full((L,), r, jnp.int32)
                acc0 = jnp.zeros((L,), jnp.float32)
                acc1 = jnp.zeros((L,), jnp.float32)
                for t in range(INPUT_DIM // L):
                    xv = x_v[slot, r, pl.ds(t * L, L)]
                    w0 = plsc.load_gather(w_v, [slot_splat, rsplat, cols[t][0]])
                    w1 = plsc.load_gather(w_v, [slot_splat, rsplat, cols[t][1]])
                    acc0 = acc0 + xv * w0
                    acc1 = acc1 + xv * w1
                outvec = jnp.where(lane_masks[2 * rr], outvec + jnp.sum(acc0), outvec)
                outvec = jnp.where(lane_masks[2 * rr + 1], outvec + jnp.sum(acc1), outvec)
            out_v[pl.ds(ci * C * 2 + g * L, L)] = outvec
            return carry

        lax.fori_loop(0, C // 8, group_body, 0)

    pltpu.sync_copy(out_v, out_hbm.at[pl.ds(base * 2, BPW * 2)])


@jax.jit
def _run(X, ids, W_embs, b_embs):
    mesh = plsc.VectorSubcoreMesh(core_axis_name="c", subcore_axis_name="s",
                                  num_cores=NC, num_subcores=NS)
    f = functools.partial(
        pl.kernel,
        out_type=jax.ShapeDtypeStruct((BATCH * 2,), jnp.float32),
        mesh=mesh,
        compiler_params=pltpu.CompilerParams(needs_layout_passes=False),
        scratch_types=[
            pltpu.VMEM((NCHUNK, C), jnp.int32),
            pltpu.VMEM((2, C, D2), jnp.float32),
            pltpu.VMEM((2, C, INPUT_DIM), jnp.float32),
            pltpu.VMEM((2, C, 128), jnp.float32),
            pltpu.VMEM((NCHUNK, C), jnp.int32),
            pltpu.VMEM((BPW * 2,), jnp.float32),
            [pltpu.SemaphoreType.DMA((2,))] * 3,
        ],
    )(_body)
    return f(X, ids, W_embs, b_embs)


def kernel(X, aspect_ids, W_embs, b_embs):
    bflat = b_embs.reshape(-1)
    pad = (-bflat.shape[0]) % 128
    b2 = jnp.pad(bflat, (0, pad)).reshape(-1, 128)
    return _run(X, aspect_ids.astype(jnp.int32), W_embs, b2).reshape(BATCH, 2)
